# baseline (speedup 1.0000x reference)
import jax
import jax.numpy as jnp
from jax import lax
from jax.experimental import pallas as pl
from jax.experimental.pallas import tpu as pltpu
from jax.experimental.pallas import tpu_sc as plsc

B, L, C = 64, 2048, 256
N = B * L
NW = 32
ROWS_W = N // NW
CHUNK = 64
NCHJ = ROWS_W // CHUNK
NBUF = 4


def _prep_body(raw_aa, raw_na, w_aa, w_na, rtype, ttype, rna, tbl, idx_aa, idx_na):
    def _norm(x, w):
        ms = jnp.mean(x * x, axis=-1, keepdims=True)
        return x * lax.rsqrt(ms + 1e-6) * w

    tbl[0:32] = _norm(raw_aa[...], w_aa[...])
    tbl[32:64] = _norm(raw_na[...], w_na[...])
    idx_aa[...] = rtype[...]
    idx_na[...] = ttype[...] + 16 * rna[...] + 32


_prep = pl.pallas_call(
    _prep_body,
    out_shape=(
        jax.ShapeDtypeStruct((64, C), jnp.float32),
        jax.ShapeDtypeStruct((B, L), jnp.int32),
        jax.ShapeDtypeStruct((B, L), jnp.int32),
    ),
)


def _sc_body(tbl, idx_aa, idx_na, out_aa, out_na, tblv, idx_v, rows_v, *ssems):
    wid = lax.axis_index("s") * 2 + lax.axis_index("c")
    base = wid * ROWS_W
    pltpu.sync_copy(tbl, tblv)
    pltpu.sync_copy(idx_aa.at[pl.ds(base, ROWS_W)], idx_v.at[pl.ds(0, ROWS_W)])
    pltpu.sync_copy(idx_na.at[pl.ds(base, ROWS_W)],
                    idx_v.at[pl.ds(ROWS_W, ROWS_W)])
    cols = [lax.iota(jnp.int32, 16) + 16 * k for k in range(16)]

    for j, out in ((0, out_aa), (1, out_na)):
        @pl.loop(0, NCHJ, step=NBUF)
        def _grp(c):
            for p in range(NBUF):
                cc = c + p
                obase = p * (CHUNK * C)

                def _wait_store():
                    pltpu.make_async_copy(
                        rows_v.at[pl.ds(obase, CHUNK * C)],
                        out.at[pl.ds(0, CHUNK * C)], ssems[p]).wait()

                if j == 0:
                    @pl.when(cc >= NBUF)
                    def _free():
                        _wait_store()
                else:
                    _wait_store()

                ibase = j * ROWS_W + cc * CHUNK

                @pl.loop(0, CHUNK, unroll=4)
                def _row(i):
                    rvec = plsc.load_gather(
                        idx_v, [jnp.zeros((16,), jnp.int32) + (ibase + i)])
                    rbase = rvec * C
                    for k in range(16):
                        val = plsc.load_gather(tblv, [rbase + cols[k]])
                        rows_v[pl.ds(obase + i * C + 16 * k, 16)] = val

                pltpu.async_copy(
                    rows_v.at[pl.ds(obase, CHUNK * C)],
                    out.at[pl.ds((base + cc * CHUNK) * C, CHUNK * C)],
                    ssems[p])
    for p in range(NBUF):
        pltpu.make_async_copy(
            rows_v.at[pl.ds(p * CHUNK * C, CHUNK * C)],
            out_na.at[pl.ds(0, CHUNK * C)], ssems[p]).wait()


_sc_gather = pl.kernel(
    _sc_body,
    out_type=(
        jax.ShapeDtypeStruct((N * C,), jnp.float32),
        jax.ShapeDtypeStruct((N * C,), jnp.float32),
    ),
    mesh=plsc.VectorSubcoreMesh(core_axis_name="c", subcore_axis_name="s"),
    compiler_params=pltpu.CompilerParams(needs_layout_passes=False),
    scratch_types=[
        pltpu.VMEM((64 * C,), jnp.float32),
        pltpu.VMEM((2 * ROWS_W,), jnp.int32),
        pltpu.VMEM((NBUF * CHUNK * C,), jnp.float32),
    ] + [pltpu.SemaphoreType.DMA] * NBUF,
)


def kernel(rtype_aa, ttype_na, tidx_na, rna, table_aa, table_na, table_type, w_aa_norm, w_na_norm):
    raw_na = jnp.concatenate(
        [jnp.tile(table_na, (2, 1)), jnp.repeat(table_type, 16, axis=0)], axis=1)
    tbl, idx_aa, idx_na = _prep(
        table_aa, raw_na, w_aa_norm.reshape(1, C), w_na_norm.reshape(1, C),
        rtype_aa, ttype_na, rna.reshape(B, 1))
    out_aa, out_na = _sc_gather(
        tbl.reshape(64 * C), idx_aa.reshape(N), idx_na.reshape(N))
    return (out_na.reshape(B, L, C), out_aa.reshape(B, L, C))


# speedup vs baseline: 1.5857x; 1.5857x over previous
import jax
import jax.numpy as jnp
from jax import lax
from jax.experimental import pallas as pl
from jax.experimental.pallas import tpu as pltpu
from jax.experimental.pallas import tpu_sc as plsc

B, L, C = 64, 2048, 256
N = B * L
NW = 32
ROWS_W = N // NW
CHUNK = 64
NCHJ = ROWS_W // CHUNK
NBUF = 4


def _prep_body(raw_aa, raw_na, w_aa, w_na, rtype, ttype, rna, tbl, idx_aa, idx_na):
    def _norm(x, w):
        ms = jnp.mean(x * x, axis=-1, keepdims=True)
        return x * lax.rsqrt(ms + 1e-6) * w

    tbl[0:32] = _norm(raw_aa[...], w_aa[...])
    tbl[32:64] = _norm(raw_na[...], w_na[...])
    idx_aa[...] = rtype[...]
    idx_na[...] = ttype[...] + 16 * rna[...] + 32


_prep = pl.pallas_call(
    _prep_body,
    out_shape=(
        jax.ShapeDtypeStruct((64, C), jnp.float32),
        jax.ShapeDtypeStruct((B, L), jnp.int32),
        jax.ShapeDtypeStruct((B, L), jnp.int32),
    ),
)


def _sc_body(tbl, idx_aa, idx_na, out_aa, out_na, tblv, idx_v, rows_v, *ssems):
    wid = lax.axis_index("s") * 2 + lax.axis_index("c")
    base = wid * ROWS_W
    pltpu.sync_copy(tbl, tblv)
    pltpu.sync_copy(idx_aa.at[pl.ds(base, ROWS_W)], idx_v.at[pl.ds(0, ROWS_W)])
    pltpu.sync_copy(idx_na.at[pl.ds(base, ROWS_W)],
                    idx_v.at[pl.ds(ROWS_W, ROWS_W)])
    cols = [lax.iota(jnp.int32, 16) + 16 * k for k in range(16)]

    for j, out in ((0, out_aa), (1, out_na)):
        @pl.loop(0, NCHJ, step=NBUF)
        def _grp(c):
            for p in range(NBUF):
                cc = c + p
                obase = p * (CHUNK * C)

                def _wait_store():
                    pltpu.make_async_copy(
                        rows_v.at[pl.ds(obase, CHUNK * C)],
                        out.at[pl.ds(0, CHUNK * C)], ssems[p]).wait()

                if j == 0:
                    @pl.when(cc >= NBUF)
                    def _free():
                        _wait_store()
                else:
                    _wait_store()

                ibase = j * ROWS_W + cc * CHUNK

                @pl.loop(0, CHUNK, unroll=4)
                def _row(i):
                    rvec = plsc.load_gather(
                        idx_v, [jnp.zeros((16,), jnp.int32) + (ibase + i)])
                    rbase = rvec * C
                    vals = [plsc.load_gather(tblv, [rbase + cols[k]])
                            for k in range(16)]
                    for k in range(16):
                        rows_v[pl.ds(obase + i * C + 16 * k, 16)] = vals[k]

                pltpu.async_copy(
                    rows_v.at[pl.ds(obase, CHUNK * C)],
                    out.at[pl.ds((base + cc * CHUNK) * C, CHUNK * C)],
                    ssems[p])
    for p in range(NBUF):
        pltpu.make_async_copy(
            rows_v.at[pl.ds(p * CHUNK * C, CHUNK * C)],
            out_na.at[pl.ds(0, CHUNK * C)], ssems[p]).wait()


_sc_gather = pl.kernel(
    _sc_body,
    out_type=(
        jax.ShapeDtypeStruct((N * C,), jnp.float32),
        jax.ShapeDtypeStruct((N * C,), jnp.float32),
    ),
    mesh=plsc.VectorSubcoreMesh(core_axis_name="c", subcore_axis_name="s"),
    compiler_params=pltpu.CompilerParams(needs_layout_passes=False),
    scratch_types=[
        pltpu.VMEM((64 * C,), jnp.float32),
        pltpu.VMEM((2 * ROWS_W,), jnp.int32),
        pltpu.VMEM((NBUF * CHUNK * C,), jnp.float32),
    ] + [pltpu.SemaphoreType.DMA] * NBUF,
)


def kernel(rtype_aa, ttype_na, tidx_na, rna, table_aa, table_na, table_type, w_aa_norm, w_na_norm):
    raw_na = jnp.concatenate(
        [jnp.tile(table_na, (2, 1)), jnp.repeat(table_type, 16, axis=0)], axis=1)
    tbl, idx_aa, idx_na = _prep(
        table_aa, raw_na, w_aa_norm.reshape(1, C), w_na_norm.reshape(1, C),
        rtype_aa, ttype_na, rna.reshape(B, 1))
    out_aa, out_na = _sc_gather(
        tbl.reshape(64 * C), idx_aa.reshape(N), idx_na.reshape(N))
    return (out_na.reshape(B, L, C), out_aa.reshape(B, L, C))


# speedup vs baseline: 3.7517x; 2.3660x over previous
import jax
import jax.numpy as jnp
from jax import lax
from jax.experimental import pallas as pl
from jax.experimental.pallas import tpu as pltpu
from jax.experimental.pallas import tpu_sc as plsc

B, L, C = 64, 2048, 256
N = B * L
NW = 32
ROWS_W = N // NW
CHUNK = 64
NCHJ = ROWS_W // CHUNK
NBUF = 4
NREP = 256


def _prep_body(raw_aa, raw_na, w_aa, w_na, rtype, ttype, rna, tbl, idx_aa, idx_na):
    def _norm(x, w):
        ms = jnp.mean(x * x, axis=-1, keepdims=True)
        return x * lax.rsqrt(ms + 1e-6) * w

    naa = _norm(raw_aa[...], w_aa[...])
    nna = _norm(raw_na[...], w_na[...])
    for r in range(NREP):
        tbl[64 * r:64 * r + 32] = naa
        tbl[64 * r + 32:64 * r + 64] = nna
    l_ids = lax.broadcasted_iota(jnp.int32, (B, L), 1)
    b_ids = lax.broadcasted_iota(jnp.int32, (B, L), 0)
    rep_off = 64 * ((b_ids + l_ids) % NREP)
    idx_aa[...] = rtype[...] + rep_off
    idx_na[...] = ttype[...] + 16 * rna[...] + 32 + rep_off


_prep = pl.pallas_call(
    _prep_body,
    out_shape=(
        jax.ShapeDtypeStruct((64 * NREP, C), jnp.float32),
        jax.ShapeDtypeStruct((B, L), jnp.int32),
        jax.ShapeDtypeStruct((B, L), jnp.int32),
    ),
)


def _sc_body(tbl, idx_aa, idx_na, out_aa, out_na, idx_v, rows_v, *sems):
    gsems, ssems = sems[:NBUF], sems[NBUF:]
    wid = lax.axis_index("s") * 2 + lax.axis_index("c")
    base = wid * ROWS_W
    pltpu.sync_copy(idx_aa.at[pl.ds(wid * NCHJ, NCHJ)], idx_v.at[0])
    pltpu.sync_copy(idx_na.at[pl.ds(wid * NCHJ, NCHJ)], idx_v.at[1])
    for j, out in ((0, out_aa), (1, out_na)):
        for q in range(NBUF - 1):
            pltpu.async_copy(tbl.at[idx_v.at[j, q]], rows_v.at[q], gsems[q])

        @pl.loop(0, NCHJ, step=NBUF)
        def _grp(c):
            for p in range(NBUF):
                cc = c + p
                pn = (p + NBUF - 1) % NBUF
                nxt = cc + NBUF - 1

                @pl.when(nxt < NCHJ)
                def _fire():
                    @pl.when(cc >= 1)
                    def _free():
                        pltpu.make_async_copy(
                            rows_v.at[pn], out.at[pl.ds(base, CHUNK)],
                            ssems[pn]).wait()
                    pltpu.async_copy(tbl.at[idx_v.at[j, nxt]], rows_v.at[pn],
                                     gsems[pn])

                pltpu.make_async_copy(tbl.at[idx_v.at[j, cc]], rows_v.at[p],
                                      gsems[p]).wait()
                pltpu.async_copy(rows_v.at[p],
                                 out.at[pl.ds(base + cc * CHUNK, CHUNK)],
                                 ssems[p])
        for p in range(NBUF):
            pltpu.make_async_copy(rows_v.at[p], out.at[pl.ds(base, CHUNK)],
                                  ssems[p]).wait()


_sc_gather = pl.kernel(
    _sc_body,
    out_type=(
        jax.ShapeDtypeStruct((N, C), jnp.float32),
        jax.ShapeDtypeStruct((N, C), jnp.float32),
    ),
    mesh=plsc.VectorSubcoreMesh(core_axis_name="c", subcore_axis_name="s"),
    scratch_types=[
        pltpu.VMEM((2, NCHJ, CHUNK), jnp.int32),
        pltpu.VMEM((NBUF, CHUNK, C), jnp.float32),
    ] + [pltpu.SemaphoreType.DMA] * (2 * NBUF),
)


def kernel(rtype_aa, ttype_na, tidx_na, rna, table_aa, table_na, table_type, w_aa_norm, w_na_norm):
    raw_na = jnp.concatenate(
        [jnp.tile(table_na, (2, 1)), jnp.repeat(table_type, 16, axis=0)], axis=1)
    tbl, idx_aa, idx_na = _prep(
        table_aa, raw_na, w_aa_norm.reshape(1, C), w_na_norm.reshape(1, C),
        rtype_aa, ttype_na, rna.reshape(B, 1))
    out_aa, out_na = _sc_gather(
        tbl, idx_aa.reshape(N // CHUNK, CHUNK), idx_na.reshape(N // CHUNK, CHUNK))
    return (out_na.reshape(B, L, C), out_aa.reshape(B, L, C))


# speedup vs baseline: 3.7824x; 1.0082x over previous
import jax
import jax.numpy as jnp
from jax import lax
from jax.experimental import pallas as pl
from jax.experimental.pallas import tpu as pltpu
from jax.experimental.pallas import tpu_sc as plsc

B, L, C = 64, 2048, 256
N = B * L
NW = 32
ROWS_W = N // NW
CHUNK = 128
NCHJ = ROWS_W // CHUNK
NBUF = 2
NREP = 256


def _prep_body(raw_aa, raw_na, w_aa, w_na, rtype, ttype, rna, tbl, idx_aa, idx_na):
    def _norm(x, w):
        ms = jnp.mean(x * x, axis=-1, keepdims=True)
        return x * lax.rsqrt(ms + 1e-6) * w

    naa = _norm(raw_aa[...], w_aa[...])
    nna = _norm(raw_na[...], w_na[...])
    for r in range(NREP):
        tbl[64 * r:64 * r + 32] = naa
        tbl[64 * r + 32:64 * r + 64] = nna
    l_ids = lax.broadcasted_iota(jnp.int32, (B, L), 1)
    b_ids = lax.broadcasted_iota(jnp.int32, (B, L), 0)
    rep_off = 64 * ((b_ids + l_ids) % NREP)
    idx_aa[...] = rtype[...] + rep_off
    idx_na[...] = ttype[...] + 16 * rna[...] + 32 + rep_off


_prep = pl.pallas_call(
    _prep_body,
    out_shape=(
        jax.ShapeDtypeStruct((64 * NREP, C), jnp.float32),
        jax.ShapeDtypeStruct((B, L), jnp.int32),
        jax.ShapeDtypeStruct((B, L), jnp.int32),
    ),
)


def _sc_body(tbl, idx_aa, idx_na, out_aa, out_na, idx_v, rows_v, *sems):
    gsems, ssems = sems[:NBUF], sems[NBUF:]
    wid = lax.axis_index("s") * 2 + lax.axis_index("c")
    base = wid * ROWS_W
    pltpu.sync_copy(idx_aa.at[pl.ds(wid * NCHJ, NCHJ)], idx_v.at[0])
    pltpu.sync_copy(idx_na.at[pl.ds(wid * NCHJ, NCHJ)], idx_v.at[1])
    for j, out in ((0, out_aa), (1, out_na)):
        for q in range(NBUF - 1):
            pltpu.async_copy(tbl.at[idx_v.at[j, q]], rows_v.at[q], gsems[q])

        @pl.loop(0, NCHJ, step=NBUF)
        def _grp(c):
            for p in range(NBUF):
                cc = c + p
                pn = (p + NBUF - 1) % NBUF
                nxt = cc + NBUF - 1

                @pl.when(nxt < NCHJ)
                def _fire():
                    @pl.when(cc >= 1)
                    def _free():
                        pltpu.make_async_copy(
                            rows_v.at[pn], out.at[pl.ds(base, CHUNK)],
                            ssems[pn]).wait()
                    pltpu.async_copy(tbl.at[idx_v.at[j, nxt]], rows_v.at[pn],
                                     gsems[pn])

                pltpu.make_async_copy(tbl.at[idx_v.at[j, cc]], rows_v.at[p],
                                      gsems[p]).wait()
                pltpu.async_copy(rows_v.at[p],
                                 out.at[pl.ds(base + cc * CHUNK, CHUNK)],
                                 ssems[p])
        for p in range(NBUF):
            pltpu.make_async_copy(rows_v.at[p], out.at[pl.ds(base, CHUNK)],
                                  ssems[p]).wait()


_sc_gather = pl.kernel(
    _sc_body,
    out_type=(
        jax.ShapeDtypeStruct((N, C), jnp.float32),
        jax.ShapeDtypeStruct((N, C), jnp.float32),
    ),
    mesh=plsc.VectorSubcoreMesh(core_axis_name="c", subcore_axis_name="s"),
    scratch_types=[
        pltpu.VMEM((2, NCHJ, CHUNK), jnp.int32),
        pltpu.VMEM((NBUF, CHUNK, C), jnp.float32),
    ] + [pltpu.SemaphoreType.DMA] * (2 * NBUF),
)


def kernel(rtype_aa, ttype_na, tidx_na, rna, table_aa, table_na, table_type, w_aa_norm, w_na_norm):
    raw_na = jnp.concatenate(
        [jnp.tile(table_na, (2, 1)), jnp.repeat(table_type, 16, axis=0)], axis=1)
    tbl, idx_aa, idx_na = _prep(
        table_aa, raw_na, w_aa_norm.reshape(1, C), w_na_norm.reshape(1, C),
        rtype_aa, ttype_na, rna.reshape(B, 1))
    out_aa, out_na = _sc_gather(
        tbl, idx_aa.reshape(N // CHUNK, CHUNK), idx_na.reshape(N // CHUNK, CHUNK))
    return (out_na.reshape(B, L, C), out_aa.reshape(B, L, C))


# speedup vs baseline: 4.2995x; 1.1367x over previous
import jax
import jax.numpy as jnp
from jax import lax
from jax.experimental import pallas as pl
from jax.experimental.pallas import tpu as pltpu
from jax.experimental.pallas import tpu_sc as plsc

B, L, C = 64, 2048, 256
N = B * L
NW = 32
ROWS_W = N // NW
CHUNK = 64
NCHJ = ROWS_W // CHUNK
NREP = 256
NSLOT = 6


def _prep_body(raw_aa, raw_na, w_aa, w_na, rtype, ttype, rna, tbl, idx_aa, idx_na):
    def _norm(x, w):
        ms = jnp.mean(x * x, axis=-1, keepdims=True)
        return x * lax.rsqrt(ms + 1e-6) * w

    naa = _norm(raw_aa[...], w_aa[...])
    nna = _norm(raw_na[...], w_na[...])
    for r in range(NREP):
        tbl[64 * r:64 * r + 32] = naa
        tbl[64 * r + 32:64 * r + 64] = nna
    l_ids = lax.broadcasted_iota(jnp.int32, (B, L), 1)
    b_ids = lax.broadcasted_iota(jnp.int32, (B, L), 0)
    rep_off = 64 * ((b_ids + l_ids) % NREP)
    idx_aa[...] = rtype[...] + rep_off
    idx_na[...] = ttype[...] + 16 * rna[...] + 32 + rep_off


_prep = pl.pallas_call(
    _prep_body,
    out_shape=(
        jax.ShapeDtypeStruct((64 * NREP, C), jnp.float32),
        jax.ShapeDtypeStruct((B, L), jnp.int32),
        jax.ShapeDtypeStruct((B, L), jnp.int32),
    ),
)


def _sc_body(tbl, idx_aa, idx_na, out_aa, out_na, tblv, idx_v, rows_v, *sems):
    gsems, ssems = sems[:4], sems[4:]
    wid = lax.axis_index("s") * 2 + lax.axis_index("c")
    base = wid * ROWS_W
    pltpu.sync_copy(tbl.at[pl.ds(0, 64)], tblv)
    pltpu.sync_copy(idx_aa.at[pl.ds(base, ROWS_W)], idx_v.at[pl.ds(0, ROWS_W)])
    pltpu.sync_copy(idx_na.at[pl.ds(base, ROWS_W)],
                    idx_v.at[pl.ds(ROWS_W, ROWS_W)])
    cols = [lax.iota(jnp.int32, 16) + 16 * k for k in range(16)]

    for j, out in ((0, out_aa), (1, out_na)):
        jb = j * ROWS_W

        def fire(nc, slot):
            pltpu.async_copy(
                tbl.at[idx_v.at[pl.ds(jb + nc * CHUNK, CHUNK)]],
                rows_v.at[slot], gsems[slot])

        def wait_gather(nc, slot):
            pltpu.make_async_copy(
                tbl.at[idx_v.at[pl.ds(jb + nc * CHUNK, CHUNK)]],
                rows_v.at[slot], gsems[slot]).wait()

        def store(cc, slot):
            pltpu.async_copy(rows_v.at[slot],
                             out.at[pl.ds(base + cc * CHUNK, CHUNK)],
                             ssems[slot])

        def wait_store(slot):
            pltpu.make_async_copy(rows_v.at[slot],
                                  out.at[pl.ds(base, CHUNK)],
                                  ssems[slot]).wait()

        def build(cc, slot):
            ibase = jb + cc * CHUNK

            @pl.loop(0, CHUNK, unroll=4)
            def _row(i):
                rvec = plsc.load_gather(
                    idx_v, [jnp.zeros((16,), jnp.int32) + (ibase + i)]) & 63
                vals = [plsc.load_gather(tblv, [rvec, cols[k]])
                        for k in range(16)]
                for k in range(16):
                    rows_v[slot, i, pl.ds(16 * k, 16)] = vals[k]

        for q, slot in ((0, 0), (1, 1), (4, 2), (5, 3)):
            fire(q, slot)

        @pl.loop(0, NCHJ, step=8)
        def _grp(c):
            for h, slot in ((0, 0), (1, 1)):
                wait_gather(c + h, slot)
                store(c + h, slot)
            for h, slot in ((2, 4), (3, 5)):
                @pl.when(c >= 8)
                def _free():
                    wait_store(slot)
                build(c + h, slot)
                store(c + h, slot)
            for h, slot in ((8, 0), (9, 1)):
                @pl.when(c + h < NCHJ)
                def _refill():
                    wait_store(slot)
                    fire(c + h, slot)
            for h, slot in ((4, 2), (5, 3)):
                wait_gather(c + h, slot)
                store(c + h, slot)
            for h, slot in ((6, 4), (7, 5)):
                wait_store(slot)
                build(c + h, slot)
                store(c + h, slot)
            for h, slot in ((12, 2), (13, 3)):
                @pl.when(c + h < NCHJ)
                def _refill2():
                    wait_store(slot)
                    fire(c + h, slot)
        for slot in range(NSLOT):
            wait_store(slot)


_sc_gather = pl.kernel(
    _sc_body,
    out_type=(
        jax.ShapeDtypeStruct((N, C), jnp.float32),
        jax.ShapeDtypeStruct((N, C), jnp.float32),
    ),
    mesh=plsc.VectorSubcoreMesh(core_axis_name="c", subcore_axis_name="s"),
    compiler_params=pltpu.CompilerParams(needs_layout_passes=False),
    scratch_types=[
        pltpu.VMEM((64, C), jnp.float32),
        pltpu.VMEM((2 * ROWS_W,), jnp.int32),
        pltpu.VMEM((NSLOT, CHUNK, C), jnp.float32),
    ] + [pltpu.SemaphoreType.DMA] * (4 + NSLOT),
)


def kernel(rtype_aa, ttype_na, tidx_na, rna, table_aa, table_na, table_type, w_aa_norm, w_na_norm):
    raw_na = jnp.concatenate(
        [jnp.tile(table_na, (2, 1)), jnp.repeat(table_type, 16, axis=0)], axis=1)
    tbl, idx_aa, idx_na = _prep(
        table_aa, raw_na, w_aa_norm.reshape(1, C), w_na_norm.reshape(1, C),
        rtype_aa, ttype_na, rna.reshape(B, 1))
    out_aa, out_na = _sc_gather(tbl, idx_aa.reshape(N), idx_na.reshape(N))
    return (out_na.reshape(B, L, C), out_aa.reshape(B, L, C))
